# baseline (speedup 1.0000x reference)
import jax
import jax.numpy as jnp
from jax.experimental import pallas as pl
from jax.experimental.pallas import tpu as pltpu


def _fused_kernel(x_ref, w_ref, adj_ref, b_ref, out_ref, s_ref):
    @pl.when(pl.program_id(0) == 0)
    def _():
        s_ref[...] = jax.lax.dot(
            x_ref[...], w_ref[...], preferred_element_type=jnp.float32
        ).astype(jnp.bfloat16)

    acc = jax.lax.dot(
        adj_ref[...], s_ref[...], preferred_element_type=jnp.float32
    )
    out_ref[...] = acc + b_ref[...]


def _pick_block(n, candidates):
    for c in candidates:
        if n % c == 0:
            return c
    return n


def kernel(input, adj, W, b):
    n, d_in = input.shape
    d_out = W.shape[1]

    bm = _pick_block(n, (80, 40, 16, 8))
    out = pl.pallas_call(
        _fused_kernel,
        grid=(n // bm,),
        in_specs=[
            pl.BlockSpec((n, d_in), lambda i: (0, 0)),
            pl.BlockSpec((d_in, d_out), lambda i: (0, 0)),
            pl.BlockSpec((bm, n), lambda i: (i, 0)),
            pl.BlockSpec((1, d_out), lambda i: (0, 0)),
        ],
        out_specs=pl.BlockSpec((bm, d_out), lambda i: (i, 0)),
        out_shape=jax.ShapeDtypeStruct((n, d_out), jnp.float32),
        scratch_shapes=[pltpu.VMEM((n, d_out), jnp.bfloat16)],
    )(input, W, adj, b)
    return out


# speedup vs baseline: 1.7708x; 1.7708x over previous
import jax
import jax.numpy as jnp
from jax.experimental import pallas as pl
from jax.experimental.pallas import tpu as pltpu


def _pick_block(n, candidates):
    for c in candidates:
        if n % c == 0:
            return c
    return n


def kernel(input, adj, W, b):
    n, d_in = input.shape
    d_out = W.shape[1]

    bm = _pick_block(n, (400, 200, 80, 40, 8))
    S = 5 if n % 5 == 0 else 1
    cs = n // S
    grid = S + n // bm

    def fused(x_ref, w_ref, adj_ref, b_ref, out_ref, s_ref):
        i = pl.program_id(0)

        @pl.when(i < S)
        def _():
            s_ref[pl.ds(i * cs, cs), :] = jax.lax.dot(
                x_ref[...], w_ref[...], preferred_element_type=jnp.float32
            ).astype(jnp.bfloat16)

        @pl.when(i >= S)
        def _():
            acc = jax.lax.dot(
                adj_ref[...], s_ref[...], preferred_element_type=jnp.float32
            )
            out_ref[...] = acc + b_ref[...]

    out = pl.pallas_call(
        fused,
        grid=(grid,),
        in_specs=[
            pl.BlockSpec((cs, d_in), lambda i: (jnp.minimum(i, S - 1), 0)),
            pl.BlockSpec((d_in, d_out), lambda i: (0, 0)),
            pl.BlockSpec((bm, n), lambda i: (jnp.maximum(i - S, 0), 0)),
            pl.BlockSpec((1, d_out), lambda i: (0, 0)),
        ],
        out_specs=pl.BlockSpec((bm, d_out), lambda i: (jnp.maximum(i - S, 0), 0)),
        out_shape=jax.ShapeDtypeStruct((n, d_out), jnp.float32),
        scratch_shapes=[pltpu.VMEM((n, d_out), jnp.bfloat16)],
    )(input, W, adj, b)
    return out
